# baseline (speedup 1.0000x reference)
import jax
import jax.numpy as jnp
from jax import lax
from jax.experimental import pallas as pl
from jax.experimental.pallas import tpu as pltpu

_BV = 256
_BVM = 512


def _stats_kernel(cnt_ref, x_ref, g_ref, s_ref, xm_ref):
    j = pl.program_id(1)
    x = x_ref[...]
    bv, p, c = x.shape
    cnt = cnt_ref[0, 0, 0, :]
    mask = (lax.broadcasted_iota(jnp.int32, (bv, p), 1) < cnt[:, None])
    maskf = mask.astype(x.dtype)
    xm = (x * maskf[:, :, None]).reshape(bv * p, c)
    xm16 = xm.astype(jnp.bfloat16)
    xm_ref[...] = xm16.reshape(bv, p, c)
    g = lax.dot_general(xm16, xm16, (((0,), (0,)), ((), ())),
                        preferred_element_type=jnp.float32)
    s = jnp.sum(xm, axis=0, keepdims=True)

    @pl.when(j == 0)
    def _init():
        g_ref[...] = g[None]
        s_ref[...] = s[None]

    @pl.when(j != 0)
    def _acc():
        g_ref[...] += g[None]
        s_ref[...] += s[None]


def _finalize_kernel(g_ref, s_ref, cnt_ref, w_ref, b_ref, gamma_ref, beta_ref,
                     w2t_ref, b2_ref):
    g = g_ref[0] + g_ref[1]
    s = s_ref[0] + s_ref[1]
    w = w_ref[...]
    p_max = 32
    cnt = jnp.minimum(cnt_ref[...], p_max).astype(jnp.float32)
    inv_n = 1.0 / jnp.sum(cnt)
    wt = w.T
    mean = lax.dot_general(s, wt, (((1,), (0,)), ((), ())),
                           preferred_element_type=jnp.float32,
                           precision=lax.Precision.HIGHEST) * inv_n + b_ref[...]
    outer = lax.dot_general(s, s, (((0,), (0,)), ((), ())),
                            preferred_element_type=jnp.float32,
                            precision=lax.Precision.HIGHEST)
    cc = g - outer * inv_n
    t = lax.dot_general(cc, wt, (((1,), (0,)), ((), ())),
                        preferred_element_type=jnp.float32,
                        precision=lax.Precision.HIGHEST)
    var = jnp.sum(t * wt, axis=0, keepdims=True) * inv_n
    scale = gamma_ref[...] * lax.rsqrt(var + 1e-5)
    w2t_ref[...] = (wt * scale).astype(jnp.bfloat16)
    b2_ref[...] = (b_ref[...] - mean) * scale + beta_ref[...]


def _main_kernel(cntc_ref, xm_ref, w2t_ref, b2_ref, o_ref):
    xm = xm_ref[...]
    bv, p, c = xm.shape
    b2 = b2_ref[...]
    f = lax.dot_general(xm.reshape(bv * p, c), w2t_ref[...],
                        (((1,), (0,)), ((), ())),
                        preferred_element_type=jnp.float32)
    f = jnp.maximum(f + b2, 0.0).reshape(bv, p, -1)
    pooled = jnp.sum(f, axis=1)
    cntf = jnp.minimum(cntc_ref[...], p).astype(jnp.float32)
    rec = jnp.where(cntf > 0.0, 1.0 / jnp.maximum(cntf, 1.0), 0.0)
    corr = (p - cntf) * rec
    relu_b2 = jnp.maximum(b2, 0.0)
    o_ref[...] = pooled * rec - corr * relu_b2


def kernel(voxel_features, voxel_num_points, W, b, gamma, beta):
    v, p, c = voxel_features.shape
    o = W.shape[0]
    nb = v // _BV
    nb2 = nb // 2
    cnt = voxel_num_points.astype(jnp.int32)
    cnt4 = cnt.reshape(2, nb2, 1, _BV)
    cntm = cnt.reshape(128, v // 128)
    cntc = cnt.reshape(v, 1)
    b_r = b.reshape(1, o)
    gamma_r = gamma.reshape(1, o)
    beta_r = beta.reshape(1, o)

    g, s, xm16 = pl.pallas_call(
        _stats_kernel,
        grid=(2, nb2),
        in_specs=[
            pl.BlockSpec((1, 1, 1, _BV), lambda i, j: (i, j, 0, 0)),
            pl.BlockSpec((_BV, p, c), lambda i, j: (i * nb2 + j, 0, 0)),
        ],
        out_specs=[
            pl.BlockSpec((1, c, c), lambda i, j: (i, 0, 0)),
            pl.BlockSpec((1, 1, c), lambda i, j: (i, 0, 0)),
            pl.BlockSpec((_BV, p, c), lambda i, j: (i * nb2 + j, 0, 0)),
        ],
        out_shape=[
            jax.ShapeDtypeStruct((2, c, c), jnp.float32),
            jax.ShapeDtypeStruct((2, 1, c), jnp.float32),
            jax.ShapeDtypeStruct((v, p, c), jnp.bfloat16),
        ],
        compiler_params=pltpu.CompilerParams(
            dimension_semantics=("parallel", "arbitrary")),
    )(cnt4, voxel_features)

    w2t, b2 = pl.pallas_call(
        _finalize_kernel,
        out_shape=[
            jax.ShapeDtypeStruct((c, o), jnp.bfloat16),
            jax.ShapeDtypeStruct((1, o), jnp.float32),
        ],
    )(g, s, cntm, W, b_r, gamma_r, beta_r)

    nbm = v // _BVM
    out = pl.pallas_call(
        _main_kernel,
        grid=(nbm,),
        in_specs=[
            pl.BlockSpec((_BVM, 1), lambda i: (i, 0)),
            pl.BlockSpec((_BVM, p, c), lambda i: (i, 0, 0)),
            pl.BlockSpec((c, o), lambda i: (0, 0)),
            pl.BlockSpec((1, o), lambda i: (0, 0)),
        ],
        out_specs=pl.BlockSpec((_BVM, o), lambda i: (i, 0)),
        out_shape=jax.ShapeDtypeStruct((v, o), jnp.float32),
        compiler_params=pltpu.CompilerParams(
            dimension_semantics=("parallel",)),
    )(cntc, xm16, w2t, b2)
    return out


# speedup vs baseline: 1.1475x; 1.1475x over previous
import jax
import jax.numpy as jnp
from jax import lax
from jax.experimental import pallas as pl
from jax.experimental.pallas import tpu as pltpu

_BV = 512
_BVM = 1024


def _stats_kernel(cnt_ref, x_ref, g_ref, s_ref, xm_ref):
    j = pl.program_id(1)
    x = x_ref[...]
    bv, p, c = x.shape
    cnt = cnt_ref[0, 0, 0, :]
    mask = (lax.broadcasted_iota(jnp.int32, (bv, p), 1) < cnt[:, None])
    maskf = mask.astype(x.dtype)
    xm = (x * maskf[:, :, None]).reshape(bv * p, c)
    xm16 = xm.astype(jnp.bfloat16)
    xm_ref[...] = xm16.reshape(bv, p, c)
    g = lax.dot_general(xm16, xm16, (((0,), (0,)), ((), ())),
                        preferred_element_type=jnp.float32)
    s = jnp.sum(xm, axis=0, keepdims=True)

    @pl.when(j == 0)
    def _init():
        g_ref[...] = g[None]
        s_ref[...] = s[None]

    @pl.when(j != 0)
    def _acc():
        g_ref[...] += g[None]
        s_ref[...] += s[None]


def _finalize_kernel(g_ref, s_ref, cnt_ref, w_ref, b_ref, gamma_ref, beta_ref,
                     w2t_ref, b2_ref):
    g = g_ref[0] + g_ref[1]
    s = s_ref[0] + s_ref[1]
    w = w_ref[...]
    p_max = 32
    cnt = jnp.minimum(cnt_ref[...], p_max).astype(jnp.float32)
    inv_n = 1.0 / jnp.sum(cnt)
    wt = w.T
    mean = lax.dot_general(s, wt, (((1,), (0,)), ((), ())),
                           preferred_element_type=jnp.float32,
                           precision=lax.Precision.HIGHEST) * inv_n + b_ref[...]
    outer = lax.dot_general(s, s, (((0,), (0,)), ((), ())),
                            preferred_element_type=jnp.float32,
                            precision=lax.Precision.HIGHEST)
    cc = g - outer * inv_n
    t = lax.dot_general(cc, wt, (((1,), (0,)), ((), ())),
                        preferred_element_type=jnp.float32,
                        precision=lax.Precision.HIGHEST)
    var = jnp.sum(t * wt, axis=0, keepdims=True) * inv_n
    scale = gamma_ref[...] * lax.rsqrt(var + 1e-5)
    w2t_ref[...] = (wt * scale).astype(jnp.bfloat16)
    b2_ref[...] = (b_ref[...] - mean) * scale + beta_ref[...]


def _main_kernel(cntc_ref, xm_ref, w2t_ref, b2_ref, o_ref):
    xm = xm_ref[...]
    bv, p, c = xm.shape
    b2 = b2_ref[...]
    f = lax.dot_general(xm.reshape(bv * p, c), w2t_ref[...],
                        (((1,), (0,)), ((), ())),
                        preferred_element_type=jnp.float32)
    f = jnp.maximum(f + b2, 0.0).reshape(bv, p, -1)
    pooled = jnp.sum(f, axis=1)
    cntf = jnp.minimum(cntc_ref[...], p).astype(jnp.float32)
    rec = jnp.where(cntf > 0.0, 1.0 / jnp.maximum(cntf, 1.0), 0.0)
    corr = (p - cntf) * rec
    relu_b2 = jnp.maximum(b2, 0.0)
    o_ref[...] = pooled * rec - corr * relu_b2


def kernel(voxel_features, voxel_num_points, W, b, gamma, beta):
    v, p, c = voxel_features.shape
    o = W.shape[0]
    nb = v // _BV
    nb2 = nb // 2
    cnt = voxel_num_points.astype(jnp.int32)
    cnt4 = cnt.reshape(2, nb2, 1, _BV)
    cntm = cnt.reshape(128, v // 128)
    cntc = cnt.reshape(v, 1)
    b_r = b.reshape(1, o)
    gamma_r = gamma.reshape(1, o)
    beta_r = beta.reshape(1, o)

    g, s, xm16 = pl.pallas_call(
        _stats_kernel,
        grid=(2, nb2),
        in_specs=[
            pl.BlockSpec((1, 1, 1, _BV), lambda i, j: (i, j, 0, 0)),
            pl.BlockSpec((_BV, p, c), lambda i, j: (i * nb2 + j, 0, 0)),
        ],
        out_specs=[
            pl.BlockSpec((1, c, c), lambda i, j: (i, 0, 0)),
            pl.BlockSpec((1, 1, c), lambda i, j: (i, 0, 0)),
            pl.BlockSpec((_BV, p, c), lambda i, j: (i * nb2 + j, 0, 0)),
        ],
        out_shape=[
            jax.ShapeDtypeStruct((2, c, c), jnp.float32),
            jax.ShapeDtypeStruct((2, 1, c), jnp.float32),
            jax.ShapeDtypeStruct((v, p, c), jnp.bfloat16),
        ],
        compiler_params=pltpu.CompilerParams(
            dimension_semantics=("parallel", "arbitrary")),
    )(cnt4, voxel_features)

    w2t, b2 = pl.pallas_call(
        _finalize_kernel,
        out_shape=[
            jax.ShapeDtypeStruct((c, o), jnp.bfloat16),
            jax.ShapeDtypeStruct((1, o), jnp.float32),
        ],
    )(g, s, cntm, W, b_r, gamma_r, beta_r)

    nbm = v // _BVM
    out = pl.pallas_call(
        _main_kernel,
        grid=(nbm,),
        in_specs=[
            pl.BlockSpec((_BVM, 1), lambda i: (i, 0)),
            pl.BlockSpec((_BVM, p, c), lambda i: (i, 0, 0)),
            pl.BlockSpec((c, o), lambda i: (0, 0)),
            pl.BlockSpec((1, o), lambda i: (0, 0)),
        ],
        out_specs=pl.BlockSpec((_BVM, o), lambda i: (i, 0)),
        out_shape=jax.ShapeDtypeStruct((v, o), jnp.float32),
        compiler_params=pltpu.CompilerParams(
            dimension_semantics=("parallel",)),
    )(cntc, xm16, w2t, b2)
    return out


# speedup vs baseline: 1.1507x; 1.0028x over previous
import jax
import jax.numpy as jnp
from jax import lax
from jax.experimental import pallas as pl
from jax.experimental.pallas import tpu as pltpu

_BV = 512
_BVM = 2048


def _stats_kernel(cnt_ref, x_ref, g_ref, s_ref, xm_ref):
    j = pl.program_id(1)
    x = x_ref[...]
    bv, p, c = x.shape
    cnt = cnt_ref[0, 0, 0, :]
    mask = (lax.broadcasted_iota(jnp.int32, (bv, p), 1) < cnt[:, None])
    maskf = mask.astype(x.dtype)
    xm = (x * maskf[:, :, None]).reshape(bv * p, c)
    xm16 = xm.astype(jnp.bfloat16)
    xm_ref[...] = xm16.reshape(bv, p, c)
    g = lax.dot_general(xm16, xm16, (((0,), (0,)), ((), ())),
                        preferred_element_type=jnp.float32)
    s = jnp.sum(xm, axis=0, keepdims=True)

    @pl.when(j == 0)
    def _init():
        g_ref[...] = g[None]
        s_ref[...] = s[None]

    @pl.when(j != 0)
    def _acc():
        g_ref[...] += g[None]
        s_ref[...] += s[None]


def _finalize_kernel(g_ref, s_ref, cnt_ref, w_ref, b_ref, gamma_ref, beta_ref,
                     w2t_ref, b2_ref):
    g = g_ref[0] + g_ref[1]
    s = s_ref[0] + s_ref[1]
    w = w_ref[...]
    p_max = 32
    cnt = jnp.minimum(cnt_ref[...], p_max).astype(jnp.float32)
    inv_n = 1.0 / jnp.sum(cnt)
    wt = w.T
    mean = lax.dot_general(s, wt, (((1,), (0,)), ((), ())),
                           preferred_element_type=jnp.float32,
                           precision=lax.Precision.HIGHEST) * inv_n + b_ref[...]
    outer = lax.dot_general(s, s, (((0,), (0,)), ((), ())),
                            preferred_element_type=jnp.float32,
                            precision=lax.Precision.HIGHEST)
    cc = g - outer * inv_n
    t = lax.dot_general(cc, wt, (((1,), (0,)), ((), ())),
                        preferred_element_type=jnp.float32,
                        precision=lax.Precision.HIGHEST)
    var = jnp.sum(t * wt, axis=0, keepdims=True) * inv_n
    scale = gamma_ref[...] * lax.rsqrt(var + 1e-5)
    w2t_ref[...] = (wt * scale).astype(jnp.bfloat16)
    b2_ref[...] = (b_ref[...] - mean) * scale + beta_ref[...]


def _main_kernel(cntc_ref, xm_ref, w2t_ref, b2_ref, o_ref):
    xm = xm_ref[...]
    bv, p, c = xm.shape
    b2 = b2_ref[...]
    f = lax.dot_general(xm.reshape(bv * p, c), w2t_ref[...],
                        (((1,), (0,)), ((), ())),
                        preferred_element_type=jnp.float32)
    f = jnp.maximum(f + b2, 0.0).reshape(bv, p, -1)
    pooled = jnp.sum(f, axis=1)
    cntf = jnp.minimum(cntc_ref[...], p).astype(jnp.float32)
    rec = jnp.where(cntf > 0.0, 1.0 / jnp.maximum(cntf, 1.0), 0.0)
    corr = (p - cntf) * rec
    relu_b2 = jnp.maximum(b2, 0.0)
    o_ref[...] = pooled * rec - corr * relu_b2


def kernel(voxel_features, voxel_num_points, W, b, gamma, beta):
    v, p, c = voxel_features.shape
    o = W.shape[0]
    nb = v // _BV
    nb2 = nb // 2
    cnt = voxel_num_points.astype(jnp.int32)
    cnt4 = cnt.reshape(2, nb2, 1, _BV)
    cntm = cnt.reshape(128, v // 128)
    cntc = cnt.reshape(v, 1)
    b_r = b.reshape(1, o)
    gamma_r = gamma.reshape(1, o)
    beta_r = beta.reshape(1, o)

    g, s, xm16 = pl.pallas_call(
        _stats_kernel,
        grid=(2, nb2),
        in_specs=[
            pl.BlockSpec((1, 1, 1, _BV), lambda i, j: (i, j, 0, 0)),
            pl.BlockSpec((_BV, p, c), lambda i, j: (i * nb2 + j, 0, 0)),
        ],
        out_specs=[
            pl.BlockSpec((1, c, c), lambda i, j: (i, 0, 0)),
            pl.BlockSpec((1, 1, c), lambda i, j: (i, 0, 0)),
            pl.BlockSpec((_BV, p, c), lambda i, j: (i * nb2 + j, 0, 0)),
        ],
        out_shape=[
            jax.ShapeDtypeStruct((2, c, c), jnp.float32),
            jax.ShapeDtypeStruct((2, 1, c), jnp.float32),
            jax.ShapeDtypeStruct((v, p, c), jnp.bfloat16),
        ],
        compiler_params=pltpu.CompilerParams(
            dimension_semantics=("parallel", "arbitrary")),
    )(cnt4, voxel_features)

    w2t, b2 = pl.pallas_call(
        _finalize_kernel,
        out_shape=[
            jax.ShapeDtypeStruct((c, o), jnp.bfloat16),
            jax.ShapeDtypeStruct((1, o), jnp.float32),
        ],
    )(g, s, cntm, W, b_r, gamma_r, beta_r)

    nbm = v // _BVM
    out = pl.pallas_call(
        _main_kernel,
        grid=(nbm,),
        in_specs=[
            pl.BlockSpec((_BVM, 1), lambda i: (i, 0)),
            pl.BlockSpec((_BVM, p, c), lambda i: (i, 0, 0)),
            pl.BlockSpec((c, o), lambda i: (0, 0)),
            pl.BlockSpec((1, o), lambda i: (0, 0)),
        ],
        out_specs=pl.BlockSpec((_BVM, o), lambda i: (i, 0)),
        out_shape=jax.ShapeDtypeStruct((v, o), jnp.float32),
        compiler_params=pltpu.CompilerParams(
            dimension_semantics=("parallel",)),
    )(cntc, xm16, w2t, b2)
    return out


# speedup vs baseline: 1.2060x; 1.0481x over previous
import jax
import jax.numpy as jnp
from jax import lax
from jax.experimental import pallas as pl
from jax.experimental.pallas import tpu as pltpu

_BV = 1024
_BVM = 2048


def _stats_kernel(cnt_ref, x_ref, g_ref, s_ref, xm_ref):
    j = pl.program_id(1)
    x = x_ref[...]
    bv, p, c = x.shape
    cnt = cnt_ref[0, 0, 0, :]
    mask = (lax.broadcasted_iota(jnp.int32, (bv, p), 1) < cnt[:, None])
    maskf = mask.astype(x.dtype)
    xm = (x * maskf[:, :, None]).reshape(bv * p, c)
    xm16 = xm.astype(jnp.bfloat16)
    xm_ref[...] = xm16.reshape(bv, p, c)
    g = lax.dot_general(xm16, xm16, (((0,), (0,)), ((), ())),
                        preferred_element_type=jnp.float32)
    s = jnp.sum(xm, axis=0, keepdims=True)

    @pl.when(j == 0)
    def _init():
        g_ref[...] = g[None]
        s_ref[...] = s[None]

    @pl.when(j != 0)
    def _acc():
        g_ref[...] += g[None]
        s_ref[...] += s[None]


def _finalize_kernel(g_ref, s_ref, cnt_ref, w_ref, b_ref, gamma_ref, beta_ref,
                     w2t_ref, b2_ref):
    g = g_ref[0] + g_ref[1]
    s = s_ref[0] + s_ref[1]
    w = w_ref[...]
    p_max = 32
    cnt = jnp.minimum(cnt_ref[...], p_max).astype(jnp.float32)
    inv_n = 1.0 / jnp.sum(cnt)
    wt = w.T
    mean = lax.dot_general(s, wt, (((1,), (0,)), ((), ())),
                           preferred_element_type=jnp.float32,
                           precision=lax.Precision.HIGHEST) * inv_n + b_ref[...]
    outer = lax.dot_general(s, s, (((0,), (0,)), ((), ())),
                            preferred_element_type=jnp.float32,
                            precision=lax.Precision.HIGHEST)
    cc = g - outer * inv_n
    t = lax.dot_general(cc, wt, (((1,), (0,)), ((), ())),
                        preferred_element_type=jnp.float32,
                        precision=lax.Precision.HIGHEST)
    var = jnp.sum(t * wt, axis=0, keepdims=True) * inv_n
    scale = gamma_ref[...] * lax.rsqrt(var + 1e-5)
    w2t_ref[...] = (wt * scale).astype(jnp.bfloat16)
    b2_ref[...] = (b_ref[...] - mean) * scale + beta_ref[...]


def _main_kernel(cntc_ref, xm_ref, w2t_ref, b2_ref, o_ref):
    xm = xm_ref[...]
    bv, p, c = xm.shape
    b2 = b2_ref[...]
    f = lax.dot_general(xm.reshape(bv * p, c), w2t_ref[...],
                        (((1,), (0,)), ((), ())),
                        preferred_element_type=jnp.float32)
    f = jnp.maximum(f + b2, 0.0).reshape(bv, p, -1)
    pooled = jnp.sum(f, axis=1)
    cntf = jnp.minimum(cntc_ref[...], p).astype(jnp.float32)
    rec = jnp.where(cntf > 0.0, 1.0 / jnp.maximum(cntf, 1.0), 0.0)
    corr = (p - cntf) * rec
    relu_b2 = jnp.maximum(b2, 0.0)
    o_ref[...] = pooled * rec - corr * relu_b2


def kernel(voxel_features, voxel_num_points, W, b, gamma, beta):
    v, p, c = voxel_features.shape
    o = W.shape[0]
    nb = v // _BV
    nb2 = nb // 2
    cnt = voxel_num_points.astype(jnp.int32)
    cnt4 = cnt.reshape(2, nb2, 1, _BV)
    cntm = cnt.reshape(128, v // 128)
    cntc = cnt.reshape(v, 1)
    b_r = b.reshape(1, o)
    gamma_r = gamma.reshape(1, o)
    beta_r = beta.reshape(1, o)

    g, s, xm16 = pl.pallas_call(
        _stats_kernel,
        grid=(2, nb2),
        in_specs=[
            pl.BlockSpec((1, 1, 1, _BV), lambda i, j: (i, j, 0, 0)),
            pl.BlockSpec((_BV, p, c), lambda i, j: (i * nb2 + j, 0, 0)),
        ],
        out_specs=[
            pl.BlockSpec((1, c, c), lambda i, j: (i, 0, 0)),
            pl.BlockSpec((1, 1, c), lambda i, j: (i, 0, 0)),
            pl.BlockSpec((_BV, p, c), lambda i, j: (i * nb2 + j, 0, 0)),
        ],
        out_shape=[
            jax.ShapeDtypeStruct((2, c, c), jnp.float32),
            jax.ShapeDtypeStruct((2, 1, c), jnp.float32),
            jax.ShapeDtypeStruct((v, p, c), jnp.bfloat16),
        ],
        compiler_params=pltpu.CompilerParams(
            dimension_semantics=("parallel", "arbitrary")),
    )(cnt4, voxel_features)

    w2t, b2 = pl.pallas_call(
        _finalize_kernel,
        out_shape=[
            jax.ShapeDtypeStruct((c, o), jnp.bfloat16),
            jax.ShapeDtypeStruct((1, o), jnp.float32),
        ],
    )(g, s, cntm, W, b_r, gamma_r, beta_r)

    nbm = v // _BVM
    out = pl.pallas_call(
        _main_kernel,
        grid=(nbm,),
        in_specs=[
            pl.BlockSpec((_BVM, 1), lambda i: (i, 0)),
            pl.BlockSpec((_BVM, p, c), lambda i: (i, 0, 0)),
            pl.BlockSpec((c, o), lambda i: (0, 0)),
            pl.BlockSpec((1, o), lambda i: (0, 0)),
        ],
        out_specs=pl.BlockSpec((_BVM, o), lambda i: (i, 0)),
        out_shape=jax.ShapeDtypeStruct((v, o), jnp.float32),
        compiler_params=pltpu.CompilerParams(
            dimension_semantics=("parallel",)),
    )(cntc, xm16, w2t, b2)
    return out
